# Initial kernel scaffold; baseline (speedup 1.0000x reference)
import jax
import jax.numpy as jnp
from jax.experimental import pallas as pl


def kernel(p3, p4, p5, targets_boxes, targets_labels):
    raise NotImplementedError("write your pallas kernel here")


# speedup vs baseline: 2.3724x; 2.3724x over previous
import jax
import jax.numpy as jnp
from jax import lax
from jax.experimental import pallas as pl
from jax.experimental.pallas import tpu as pltpu

_C = 80
_SCALES = ((52, 52), (26, 26), (13, 13))
_B = 32
_G = 20
_A = 3


def _softplus(x):
    return jnp.maximum(x, 0.0) + jnp.log1p(jnp.exp(-jnp.abs(x)))


def _loss_kernel(o3, o4, o5, s3, s4, s5, cr3, cr4, cr5, cc3, cc4, cc5,
                 lr, lc, bx, out_ref, acc_ref):
    b = pl.program_id(0)

    @pl.when(b == 0)
    def _init():
        for i in range(12):
            acc_ref[i] = 0.0

    lrow = lr[0]
    lcol = lc[0]
    boxes_row = bx[0]
    gidx_r = lax.broadcasted_iota(jnp.int32, (_G, _G), 0)
    gidx_c = lax.broadcasted_iota(jnp.int32, (_G, _G), 1)
    later = gidx_c > gidx_r
    cls_iota = lax.broadcasted_iota(jnp.int32, (_G, _C), 1)
    oh_lab = (cls_iota == lcol).astype(jnp.float32)

    for s, (obj_ref, slab_ref, crow_ref, ccol_ref) in enumerate((
            (o3, s3, cr3, cc3), (o4, s4, cr4, cc4), (o5, s5, cr5, cc5))):
        H, W = _SCALES[s]
        S = H * W

        acc_ref[s] = acc_ref[s] + jnp.sum(_softplus(obj_ref[0]))

        crow = crow_ref[0]
        ccol = ccol_ref[0]
        same_cell = ccol == crow
        winner = 1.0 - jnp.max((same_cell & later).astype(jnp.float32),
                               axis=1, keepdims=True)
        same_pair = same_cell & (lcol == lrow)
        pairw = 1.0 - jnp.max((same_pair & later).astype(jnp.float32),
                              axis=1, keepdims=True)
        cnt = jnp.maximum(jnp.sum(winner), 1.0)

        slab = slab_ref[0, 0]
        pos_iota = lax.broadcasted_iota(jnp.int32, (_G, S), 1)
        onehot = (pos_iota == ccol).astype(jnp.float32)
        cv = lax.dot_general(onehot, slab, (((1,), (1,)), ((), ())),
                             preferred_element_type=jnp.float32)
        v0 = cv[:, 0:1]
        vb = cv[:, 1:5]
        vc = cv[:, 5:85]

        acc_ref[3 + s] = acc_ref[3 + s] + jnp.sum(winner * v0)

        gx = (ccol % W).astype(jnp.float32)
        gy = (ccol // W).astype(jnp.float32)
        tx = boxes_row[:, 0:1] * W - gx
        ty = boxes_row[:, 1:2] * H - gy
        tv = jnp.concatenate(
            [tx, ty, boxes_row[:, 2:3], boxes_row[:, 3:4]], axis=1)
        mse = jnp.sum((vb - tv) ** 2, axis=1, keepdims=True)
        acc_ref[6 + s] = acc_ref[6 + s] + jnp.sum(winner * mse) / (cnt * 4.0)

        spsum = jnp.sum(_softplus(vc), axis=1, keepdims=True)
        xlab = jnp.sum(vc * oh_lab, axis=1, keepdims=True)
        acc_ref[9 + s] = acc_ref[9 + s] + (
            jnp.sum(winner * spsum) - jnp.sum(pairw * xlab)) / (cnt * _C)

    @pl.when(b == pl.num_programs(0) - 1)
    def _fin():
        to = 0.0
        for s, (H, W) in enumerate(_SCALES):
            to = to + (acc_ref[s] - acc_ref[3 + s]) / (_B * _A * H * W)
        to = to / 3.0
        tb = (acc_ref[6] + acc_ref[7] + acc_ref[8]) / (_B * _G * 3.0)
        tc = (acc_ref[9] + acc_ref[10] + acc_ref[11]) / (_B * _G * 3.0)
        out_ref[0] = to + 5.0 * tb + 2.0 * tc
        out_ref[1] = to
        out_ref[2] = tb
        out_ref[3] = tc


def kernel(p3, p4, p5, targets_boxes, targets_labels):
    objs, slabs, crows, ccols = [], [], [], []
    for pred, (H, W) in zip((p3, p4, p5), _SCALES):
        S = H * W
        pr = pred.reshape(_B, _A, 85, S)
        slabs.append(pr)
        objs.append(pr[:, :, 0, :])
        cx = targets_boxes[..., 0]
        cy = targets_boxes[..., 1]
        gx = jnp.clip((cx * W).astype(jnp.int32), 0, W - 1)
        gy = jnp.clip((cy * H).astype(jnp.int32), 0, H - 1)
        cell = gy * W + gx
        crows.append(cell[:, None, :])
        ccols.append(cell[:, :, None])
    labs = targets_labels.astype(jnp.int32)
    lr = labs[:, None, :]
    lc = labs[:, :, None]

    in_specs = []
    for (H, W) in _SCALES:
        in_specs.append(pl.BlockSpec((1, _A, H * W), lambda b: (b, 0, 0)))
    for (H, W) in _SCALES:
        in_specs.append(pl.BlockSpec((1, 1, 85, H * W), lambda b: (b, 0, 0, 0)))
    for _ in range(3):
        in_specs.append(pl.BlockSpec((1, 1, _G), lambda b: (b, 0, 0)))
    for _ in range(3):
        in_specs.append(pl.BlockSpec((1, _G, 1), lambda b: (b, 0, 0)))
    in_specs.append(pl.BlockSpec((1, 1, _G), lambda b: (b, 0, 0)))
    in_specs.append(pl.BlockSpec((1, _G, 1), lambda b: (b, 0, 0)))
    in_specs.append(pl.BlockSpec((1, _G, 4), lambda b: (b, 0, 0)))

    out = pl.pallas_call(
        _loss_kernel,
        grid=(_B,),
        in_specs=in_specs,
        out_specs=pl.BlockSpec(memory_space=pltpu.SMEM),
        out_shape=jax.ShapeDtypeStruct((4,), jnp.float32),
        scratch_shapes=[pltpu.SMEM((12,), jnp.float32)],
        compiler_params=pltpu.CompilerParams(
            dimension_semantics=("arbitrary",)),
    )(*objs, *slabs, *crows, *ccols, lr, lc, targets_boxes)
    return (out[0], out[1], out[2], out[3])


# speedup vs baseline: 4.4531x; 1.8771x over previous
import jax
import jax.numpy as jnp
from jax import lax
from jax.experimental import pallas as pl
from jax.experimental.pallas import tpu as pltpu

_C = 80
_SCALES = ((52, 52), (26, 26), (13, 13))
_B = 32
_G = 20
_A = 3


def _softplus(x):
    return jnp.maximum(x, 0.0) + jnp.log1p(jnp.exp(-jnp.abs(x)))


def _loss_kernel(oa3, ob3, oa4, ob4, oa5, ob5, s3, s4, s5,
                 cr3, cr4, cr5, cc3, cc4, cc5,
                 lr, lc, bx, out_ref, acc_ref):
    b = pl.program_id(0)

    @pl.when(b == 0)
    def _init():
        for i in range(12):
            acc_ref[i] = 0.0

    lrow = lr[0]
    lcol = lc[0]
    boxes_row = bx[0]
    gidx_r = lax.broadcasted_iota(jnp.int32, (_G, _G), 0)
    gidx_c = lax.broadcasted_iota(jnp.int32, (_G, _G), 1)
    later = gidx_c > gidx_r
    cls_iota = lax.broadcasted_iota(jnp.int32, (_G, _C), 1)
    oh_lab = (cls_iota == lcol).astype(jnp.float32)

    for s, (obj1_ref, obj2_ref, slab_ref, crow_ref, ccol_ref) in enumerate((
            (oa3, ob3, s3, cr3, cc3), (oa4, ob4, s4, cr4, cc4),
            (oa5, ob5, s5, cr5, cc5))):
        H, W = _SCALES[s]
        S = H * W
        slab = slab_ref[0]

        acc_ref[s] = (acc_ref[s] + jnp.sum(_softplus(slab[0:1, :]))
                      + jnp.sum(_softplus(obj1_ref[0, 0]))
                      + jnp.sum(_softplus(obj2_ref[0, 0])))

        crow = crow_ref[0]
        ccol = ccol_ref[0]
        same_cell = ccol == crow
        winner = 1.0 - jnp.max((same_cell & later).astype(jnp.float32),
                               axis=1, keepdims=True)
        same_pair = same_cell & (lcol == lrow)
        pairw = 1.0 - jnp.max((same_pair & later).astype(jnp.float32),
                              axis=1, keepdims=True)
        cnt = jnp.maximum(jnp.sum(winner), 1.0)

        pos_iota = lax.broadcasted_iota(jnp.int32, (_G, S), 1)
        onehot = (pos_iota == ccol).astype(jnp.float32)
        cv = lax.dot_general(onehot, slab, (((1,), (1,)), ((), ())),
                             preferred_element_type=jnp.float32)
        v0 = cv[:, 0:1]
        vb = cv[:, 1:5]
        vc = cv[:, 5:85]

        acc_ref[3 + s] = acc_ref[3 + s] + jnp.sum(winner * v0)

        gx = (ccol % W).astype(jnp.float32)
        gy = (ccol // W).astype(jnp.float32)
        tx = boxes_row[:, 0:1] * W - gx
        ty = boxes_row[:, 1:2] * H - gy
        tv = jnp.concatenate(
            [tx, ty, boxes_row[:, 2:3], boxes_row[:, 3:4]], axis=1)
        mse = jnp.sum((vb - tv) ** 2, axis=1, keepdims=True)
        acc_ref[6 + s] = acc_ref[6 + s] + jnp.sum(winner * mse) / (cnt * 4.0)

        spsum = jnp.sum(_softplus(vc), axis=1, keepdims=True)
        xlab = jnp.sum(vc * oh_lab, axis=1, keepdims=True)
        acc_ref[9 + s] = acc_ref[9 + s] + (
            jnp.sum(winner * spsum) - jnp.sum(pairw * xlab)) / (cnt * _C)

    @pl.when(b == pl.num_programs(0) - 1)
    def _fin():
        to = 0.0
        for s, (H, W) in enumerate(_SCALES):
            to = to + (acc_ref[s] - acc_ref[3 + s]) / (_B * _A * H * W)
        to = to / 3.0
        tb = (acc_ref[6] + acc_ref[7] + acc_ref[8]) / (_B * _G * 3.0)
        tc = (acc_ref[9] + acc_ref[10] + acc_ref[11]) / (_B * _G * 3.0)
        out_ref[0] = to + 5.0 * tb + 2.0 * tc
        out_ref[1] = to
        out_ref[2] = tb
        out_ref[3] = tc


def kernel(p3, p4, p5, targets_boxes, targets_labels):
    objs, obj_specs, slabs, crows, ccols = [], [], [], [], []
    for pred, (H, W) in zip((p3, p4, p5), _SCALES):
        S = H * W
        slabs.append(pred[:, :85].reshape(_B, 85, S))
        objs.extend([pred, pred])
        obj_specs.append(pl.BlockSpec((1, 1, H, W), lambda b: (b, 85, 0, 0)))
        obj_specs.append(pl.BlockSpec((1, 1, H, W), lambda b: (b, 170, 0, 0)))
        cx = targets_boxes[..., 0]
        cy = targets_boxes[..., 1]
        gx = jnp.clip((cx * W).astype(jnp.int32), 0, W - 1)
        gy = jnp.clip((cy * H).astype(jnp.int32), 0, H - 1)
        cell = gy * W + gx
        crows.append(cell[:, None, :])
        ccols.append(cell[:, :, None])
    labs = targets_labels.astype(jnp.int32)
    lr = labs[:, None, :]
    lc = labs[:, :, None]

    in_specs = list(obj_specs)
    for (H, W) in _SCALES:
        in_specs.append(pl.BlockSpec((1, 85, H * W), lambda b: (b, 0, 0)))
    for _ in range(3):
        in_specs.append(pl.BlockSpec((1, 1, _G), lambda b: (b, 0, 0)))
    for _ in range(3):
        in_specs.append(pl.BlockSpec((1, _G, 1), lambda b: (b, 0, 0)))
    in_specs.append(pl.BlockSpec((1, 1, _G), lambda b: (b, 0, 0)))
    in_specs.append(pl.BlockSpec((1, _G, 1), lambda b: (b, 0, 0)))
    in_specs.append(pl.BlockSpec((1, _G, 4), lambda b: (b, 0, 0)))

    out = pl.pallas_call(
        _loss_kernel,
        grid=(_B,),
        in_specs=in_specs,
        out_specs=pl.BlockSpec(memory_space=pltpu.SMEM),
        out_shape=jax.ShapeDtypeStruct((4,), jnp.float32),
        scratch_shapes=[pltpu.SMEM((12,), jnp.float32)],
        compiler_params=pltpu.CompilerParams(
            dimension_semantics=("arbitrary",)),
    )(*objs, *slabs, *crows, *ccols, lr, lc, targets_boxes)
    return (out[0], out[1], out[2], out[3])
